# speedup vs baseline: 2.4059x; 2.4059x over previous
import functools

import jax
import jax.numpy as jnp
from jax import lax
from jax.experimental import pallas as pl
from jax.experimental.pallas import tpu as pltpu
from jax.experimental.pallas import tpu_sc as plsc

_EMBED = 128
_CHUNK = 128


def kernel(author_idx, publisher_idx, year_idx, W_author, W_year, W_publisher):
    batch = author_idx.shape[0]
    info = plsc.get_sparse_core_info()
    num_cores = info.num_cores
    nw = num_cores * info.num_subcores
    b_per_w = batch // nw
    n_chunks = b_per_w // _CHUNK

    mesh = plsc.VectorSubcoreMesh(core_axis_name="c", subcore_axis_name="s")

    @functools.partial(
        pl.kernel,
        out_type=jax.ShapeDtypeStruct((batch, 3 * _EMBED), jnp.float32),
        mesh=mesh,
        scratch_types=[
            pltpu.VMEM((b_per_w,), jnp.int32),
            pltpu.VMEM((_CHUNK, _EMBED), jnp.float32),
            pltpu.SemaphoreType.DMA,
        ],
    )
    def _gather3(a_idx, p_idx, y_idx, wa, wy, wp, out, idx_v, rows_v, gsem):
        wid = lax.axis_index("s") * num_cores + lax.axis_index("c")
        base = wid * b_per_w
        tables = ((wa, a_idx, 0), (wy, y_idx, _EMBED), (wp, p_idx, 2 * _EMBED))
        for tbl, idx_hbm, col in tables:
            pltpu.sync_copy(idx_hbm.at[pl.ds(base, b_per_w)], idx_v)
            for j in range(n_chunks):
                chunk_idx = idx_v.at[pl.ds(j * _CHUNK, _CHUNK)]
                pltpu.async_copy(tbl.at[chunk_idx], rows_v, gsem).wait()
                pltpu.sync_copy(
                    rows_v,
                    out.at[pl.ds(base + j * _CHUNK, _CHUNK), pl.ds(col, _EMBED)],
                )

    return _gather3(author_idx, publisher_idx, year_idx, W_author, W_year, W_publisher)


# speedup vs baseline: 2.8684x; 1.1922x over previous
import functools

import jax
import jax.numpy as jnp
from jax import lax
from jax.experimental import pallas as pl
from jax.experimental.pallas import tpu as pltpu
from jax.experimental.pallas import tpu_sc as plsc

_EMBED = 128
_CHUNK = 128
_NBUF = 3


def kernel(author_idx, publisher_idx, year_idx, W_author, W_year, W_publisher):
    batch = author_idx.shape[0]
    info = plsc.get_sparse_core_info()
    num_cores = info.num_cores
    nw = num_cores * info.num_subcores
    b_per_w = batch // nw
    n_chunks = b_per_w // _CHUNK

    mesh = plsc.VectorSubcoreMesh(core_axis_name="c", subcore_axis_name="s")

    @functools.partial(
        pl.kernel,
        out_type=jax.ShapeDtypeStruct((batch, 3 * _EMBED), jnp.float32),
        mesh=mesh,
        scratch_types=[
            pltpu.VMEM((3 * b_per_w,), jnp.int32),
            pltpu.VMEM((_NBUF, _CHUNK, _EMBED), jnp.float32),
            pltpu.SemaphoreType.DMA,
        ]
        + [pltpu.SemaphoreType.DMA] * (2 * _NBUF),
    )
    def _gather3(a_idx, p_idx, y_idx, wa, wy, wp, out, idx_v, rows_v, isem, *sems):
        gsems = sems[:_NBUF]
        wsems = sems[_NBUF:]
        wid = lax.axis_index("s") * num_cores + lax.axis_index("c")
        base = wid * b_per_w

        idx_copies = [
            pltpu.async_copy(
                src.at[pl.ds(base, b_per_w)],
                idx_v.at[pl.ds(r * b_per_w, b_per_w)],
                isem,
            )
            for r, src in enumerate((a_idx, y_idx, p_idx))
        ]
        for c in idx_copies:
            c.wait()

        tables = (wa, wy, wp)
        tasks = [
            (r, j, r * _EMBED)
            for r in range(3)
            for j in range(n_chunks)
        ]
        T = len(tasks)

        def start_gather(t):
            r, j, _ = tasks[t]
            b = t % _NBUF
            return pltpu.async_copy(
                tables[r].at[idx_v.at[pl.ds(r * b_per_w + j * _CHUNK, _CHUNK)]],
                rows_v.at[b],
                gsems[b],
            )

        def start_write(t):
            r, j, col = tasks[t]
            b = t % _NBUF
            return pltpu.async_copy(
                rows_v.at[b],
                out.at[pl.ds(base + j * _CHUNK, _CHUNK), pl.ds(col, _EMBED)],
                wsems[b],
            )

        gcp, wcp = {}, {}
        for t in range(min(_NBUF - 1, T)):
            gcp[t] = start_gather(t)
        for t in range(T):
            gcp[t].wait()
            wcp[t] = start_write(t)
            u = t + _NBUF - 1
            if u < T:
                if u - _NBUF >= 0:
                    wcp[u - _NBUF].wait()
                gcp[u] = start_gather(u)
        for t in range(max(0, T - _NBUF), T):
            wcp[t].wait()

    return _gather3(author_idx, publisher_idx, year_idx, W_author, W_year, W_publisher)


# speedup vs baseline: 2.9729x; 1.0365x over previous
import functools

import jax
import jax.numpy as jnp
from jax import lax
from jax.experimental import pallas as pl
from jax.experimental.pallas import tpu as pltpu
from jax.experimental.pallas import tpu_sc as plsc

_EMBED = 128
_CHUNK = 128
_NBUF = 6


def kernel(author_idx, publisher_idx, year_idx, W_author, W_year, W_publisher):
    batch = author_idx.shape[0]
    info = plsc.get_sparse_core_info()
    num_cores = info.num_cores
    nw = num_cores * info.num_subcores
    b_per_w = batch // nw
    n_chunks = b_per_w // _CHUNK

    mesh = plsc.VectorSubcoreMesh(core_axis_name="c", subcore_axis_name="s")

    @functools.partial(
        pl.kernel,
        out_type=jax.ShapeDtypeStruct((batch, 3 * _EMBED), jnp.float32),
        mesh=mesh,
        scratch_types=[
            pltpu.VMEM((3 * b_per_w,), jnp.int32),
            pltpu.VMEM((_NBUF, _CHUNK, _EMBED), jnp.float32),
            pltpu.SemaphoreType.DMA,
        ]
        + [pltpu.SemaphoreType.DMA] * (2 * _NBUF),
    )
    def _gather3(a_idx, p_idx, y_idx, wa, wy, wp, out, idx_v, rows_v, isem, *sems):
        gsems = sems[:_NBUF]
        wsems = sems[_NBUF:]
        wid = lax.axis_index("s") * num_cores + lax.axis_index("c")
        base = wid * b_per_w

        idx_copies = [
            pltpu.async_copy(
                src.at[pl.ds(base, b_per_w)],
                idx_v.at[pl.ds(r * b_per_w, b_per_w)],
                isem,
            )
            for r, src in enumerate((a_idx, y_idx, p_idx))
        ]
        for c in idx_copies:
            c.wait()

        tables = (wa, wy, wp)
        tasks = [
            (r, j, r * _EMBED)
            for r in range(3)
            for j in range(n_chunks)
        ]
        T = len(tasks)

        def start_gather(t):
            r, j, _ = tasks[t]
            b = t % _NBUF
            return pltpu.async_copy(
                tables[r].at[idx_v.at[pl.ds(r * b_per_w + j * _CHUNK, _CHUNK)]],
                rows_v.at[b],
                gsems[b],
            )

        def start_write(t):
            r, j, col = tasks[t]
            b = t % _NBUF
            return pltpu.async_copy(
                rows_v.at[b],
                out.at[pl.ds(base + j * _CHUNK, _CHUNK), pl.ds(col, _EMBED)],
                wsems[b],
            )

        gcp, wcp = {}, {}
        for t in range(min(_NBUF - 1, T)):
            gcp[t] = start_gather(t)
        for t in range(T):
            gcp[t].wait()
            wcp[t] = start_write(t)
            u = t + _NBUF - 1
            if u < T:
                if u - _NBUF >= 0:
                    wcp[u - _NBUF].wait()
                gcp[u] = start_gather(u)
        for t in range(max(0, T - _NBUF), T):
            wcp[t].wait()

    return _gather3(author_idx, publisher_idx, year_idx, W_author, W_year, W_publisher)
